# baseline (speedup 1.0000x reference)
import jax
import jax.numpy as jnp
import numpy as np
from jax import lax
from jax.experimental import pallas as pl
from jax.experimental.pallas import tpu as pltpu

_C = 64
_K = 16
_DEPTH = 4
_D = 512
_M = 512
_TN = 512
_SUB = 256


_W = 2 * _C


def _encode_chunk(proj, t):
    proj = proj.astype(jnp.bfloat16).astype(jnp.float32)

    def trow(i):
        return t[i, :][None, :]

    p0 = proj[:, 0 * _W:1 * _W]
    p1 = proj[:, 1 * _W:2 * _W]
    p2 = proj[:, 2 * _W:3 * _W]
    p3 = proj[:, 3 * _W:4 * _W]

    b0 = p0 > trow(0)
    t1 = jnp.where(b0, trow(2), trow(1))
    b1 = p1 > t1
    t2 = jnp.where(b0,
                   jnp.where(b1, trow(6), trow(5)),
                   jnp.where(b1, trow(4), trow(3)))
    b2 = p2 > t2
    t3 = jnp.where(
        b0,
        jnp.where(b1,
                  jnp.where(b2, trow(14), trow(13)),
                  jnp.where(b2, trow(12), trow(11))),
        jnp.where(b1,
                  jnp.where(b2, trow(10), trow(9)),
                  jnp.where(b2, trow(8), trow(7))))
    b3 = p3 > t3

    one = jnp.float32(1)
    neg = jnp.float32(-1)
    s1 = jnp.where(b1, one, neg)
    s2 = jnp.where(b2, one, neg)
    s3 = jnp.where(b3, one, neg)
    lane = lax.broadcasted_iota(jnp.int32, b0.shape, 1)
    b0x = jnp.logical_or(lane < _C, b0)
    B0 = jnp.where(b0x, one, neg)
    s12 = s1 * s2
    s13 = s1 * s3
    s23 = s2 * s3
    s123 = s12 * s3
    pairs = [B0, s1 * B0, s2 * B0, s12 * B0,
             s3 * B0, s13 * B0, s23 * B0, s123 * B0]
    return jnp.concatenate([g.astype(jnp.bfloat16) for g in pairs], axis=1)


def _fused_body(x_ref, abd_ref, tn_ref, lr_ref, o_ref):
    abd = abd_ref[...]
    t = tn_ref[...]
    nsub = _TN // _SUB
    projs = [
        jnp.dot(x_ref[pl.ds(s * _SUB, _SUB), :].astype(jnp.bfloat16), abd,
                preferred_element_type=jnp.float32)
        for s in range(nsub)
    ]
    lr = lr_ref[...]
    for s in range(nsub):
        onehot = _encode_chunk(projs[s], t)
        o_ref[pl.ds(s * _SUB, _SUB), :] = jnp.dot(
            onehot, lr, preferred_element_type=jnp.float32)


def kernel(I, T, L, S, B, A):
    del S, B
    N = I.shape[0]
    f = _D // _C

    eye = jnp.eye(_C, dtype=A.dtype)
    a4 = jnp.einsum('cjl,cx->cjlx', A, eye).reshape(_C * f, _DEPTH, _C)
    abd = jnp.concatenate([a4[:, :, None, :], a4[:, :, None, :]],
                          axis=2).reshape(_C * f, _DEPTH * _W)
    abd = abd.astype(jnp.bfloat16)

    t15 = T.reshape(_C, _K - 1).T
    tn = jnp.pad(jnp.concatenate([t15, t15], axis=1), ((0, 16 - (_K - 1)), (0, 0)))

    kk = np.arange(_K)
    hm = np.ones((_K, _K), np.float32)
    for l in range(_DEPTH):
        kbit = 2.0 * ((kk >> (_DEPTH - 1 - l)) & 1) - 1.0
        for s_mask in range(_K):
            if (s_mask >> l) & 1:
                hm[s_mask, :] *= kbit
    hm = jnp.asarray(hm / _K)
    w = jnp.einsum('mck,sk->scm', L, hm,
                   precision=lax.Precision.HIGHEST)
    lr = w.reshape(_K * _C, _M).astype(jnp.bfloat16)

    return pl.pallas_call(
        _fused_body,
        grid=(N // _TN,),
        in_specs=[
            pl.BlockSpec((_TN, _D), lambda i: (i, 0)),
            pl.BlockSpec((_D, _DEPTH * _W), lambda i: (0, 0)),
            pl.BlockSpec((16, 128), lambda i: (0, 0)),
            pl.BlockSpec((_K * _C, _M), lambda i: (0, 0)),
        ],
        out_specs=pl.BlockSpec((_TN, _M), lambda i: (i, 0)),
        out_shape=jax.ShapeDtypeStruct((N, _M), jnp.float32),
    )(I, abd, tn, lr)


# speedup vs baseline: 1.2062x; 1.2062x over previous
import jax
import jax.numpy as jnp
import numpy as np
from jax import lax
from jax.experimental import pallas as pl
from jax.experimental.pallas import tpu as pltpu

_C = 64
_K = 16
_DEPTH = 4
_D = 512
_M = 512
_TN = 2048
_SUB = 256


_W = 2 * _C


def _encode_chunk(proj, t):
    proj = proj.astype(jnp.bfloat16).astype(jnp.float32)

    def trow(i):
        return t[i, :][None, :]

    p0 = proj[:, 0 * _W:1 * _W]
    p1 = proj[:, 1 * _W:2 * _W]
    p2 = proj[:, 2 * _W:3 * _W]
    p3 = proj[:, 3 * _W:4 * _W]

    b0 = p0 > trow(0)
    t1 = jnp.where(b0, trow(2), trow(1))
    b1 = p1 > t1
    t2 = jnp.where(b0,
                   jnp.where(b1, trow(6), trow(5)),
                   jnp.where(b1, trow(4), trow(3)))
    b2 = p2 > t2
    t3 = jnp.where(
        b0,
        jnp.where(b1,
                  jnp.where(b2, trow(14), trow(13)),
                  jnp.where(b2, trow(12), trow(11))),
        jnp.where(b1,
                  jnp.where(b2, trow(10), trow(9)),
                  jnp.where(b2, trow(8), trow(7))))
    b3 = p3 > t3

    one = jnp.float32(1)
    neg = jnp.float32(-1)
    s1 = jnp.where(b1, one, neg)
    s2 = jnp.where(b2, one, neg)
    s3 = jnp.where(b3, one, neg)
    lane = lax.broadcasted_iota(jnp.int32, b0.shape, 1)
    b0x = jnp.logical_or(lane < _C, b0)
    B0 = jnp.where(b0x, one, neg)
    s12 = s1 * s2
    s13 = s1 * s3
    s23 = s2 * s3
    s123 = s12 * s3
    pairs = [B0, s1 * B0, s2 * B0, s12 * B0,
             s3 * B0, s13 * B0, s23 * B0, s123 * B0]
    return jnp.concatenate([g.astype(jnp.bfloat16) for g in pairs], axis=1)


def _fused_body(x_ref, abd_ref, tn_ref, lr_ref, o_ref):
    abd = abd_ref[...]
    t = tn_ref[...]
    nsub = _TN // _SUB
    projs = [
        jnp.dot(x_ref[pl.ds(s * _SUB, _SUB), :].astype(jnp.bfloat16), abd,
                preferred_element_type=jnp.float32)
        for s in range(nsub)
    ]
    lr = lr_ref[...]
    for s in range(nsub):
        onehot = _encode_chunk(projs[s], t)
        o_ref[pl.ds(s * _SUB, _SUB), :] = jnp.dot(
            onehot, lr, preferred_element_type=jnp.float32)


def kernel(I, T, L, S, B, A):
    del S, B
    N = I.shape[0]
    f = _D // _C

    eye = jnp.eye(_C, dtype=A.dtype)
    a4 = jnp.einsum('cjl,cx->cjlx', A, eye).reshape(_C * f, _DEPTH, _C)
    abd = jnp.concatenate([a4[:, :, None, :], a4[:, :, None, :]],
                          axis=2).reshape(_C * f, _DEPTH * _W)
    abd = abd.astype(jnp.bfloat16)

    t15 = T.reshape(_C, _K - 1).T
    tn = jnp.pad(jnp.concatenate([t15, t15], axis=1), ((0, 16 - (_K - 1)), (0, 0)))

    kk = np.arange(_K)
    hm = np.ones((_K, _K), np.float32)
    for l in range(_DEPTH):
        kbit = 2.0 * ((kk >> (_DEPTH - 1 - l)) & 1) - 1.0
        for s_mask in range(_K):
            if (s_mask >> l) & 1:
                hm[s_mask, :] *= kbit
    hm = jnp.asarray(hm / _K)
    w = jnp.einsum('mck,sk->scm', L, hm,
                   precision=lax.Precision.HIGHEST)
    lr = w.reshape(_K * _C, _M).astype(jnp.bfloat16)

    return pl.pallas_call(
        _fused_body,
        grid=(N // _TN,),
        in_specs=[
            pl.BlockSpec((_TN, _D), lambda i: (i, 0)),
            pl.BlockSpec((_D, _DEPTH * _W), lambda i: (0, 0)),
            pl.BlockSpec((16, 128), lambda i: (0, 0)),
            pl.BlockSpec((_K * _C, _M), lambda i: (0, 0)),
        ],
        out_specs=pl.BlockSpec((_TN, _M), lambda i: (i, 0)),
        out_shape=jax.ShapeDtypeStruct((N, _M), jnp.float32),
    )(I, abd, tn, lr)
